# baseline (speedup 1.0000x reference)
import jax
import jax.numpy as jnp
from jax.experimental import pallas as pl
from jax.experimental.pallas import tpu as pltpu

_CH = 128
_NROWS = 21504
_NCOPY = _NROWS // _CH


def _bcast_kernel(logps_ref, out_ref, bufa, patt, sem_stage, sem_back, sem_out):
    V = logps_ref.shape[1]
    out_rows = out_ref.reshape(_NROWS, V)
    bufa[...] = jnp.broadcast_to(logps_ref[...], bufa.shape)
    stage = pltpu.make_async_copy(bufa, out_rows.at[pl.ds(0, _CH), :], sem_stage)
    stage.start()
    stage.wait()
    back = pltpu.make_async_copy(out_rows.at[pl.ds(0, _CH), :], patt, sem_back)
    back.start()
    back.wait()
    for i in range(1, _NCOPY):
        pltpu.make_async_copy(
            patt, out_rows.at[pl.ds(i * _CH, _CH), :], sem_out
        ).start()
    for i in range(1, _NCOPY):
        pltpu.make_async_copy(
            patt, out_rows.at[pl.ds(i * _CH, _CH), :], sem_out
        ).wait()


def kernel(hist, logps):
    S, B = hist.shape
    V = logps.shape[0]
    logps2d = logps.reshape(1, V)

    out = pl.pallas_call(
        _bcast_kernel,
        in_specs=[pl.BlockSpec((1, V), lambda: (0, 0))],
        out_specs=pl.BlockSpec(memory_space=pltpu.MemorySpace.HBM),
        out_shape=jax.ShapeDtypeStruct((S + 1, B, V), jnp.float32),
        scratch_shapes=[
            pltpu.VMEM((_CH, 1000), jnp.float32),
            pltpu.SMEM((_CH, 1000), jnp.float32),
            pltpu.SemaphoreType.DMA,
            pltpu.SemaphoreType.DMA,
            pltpu.SemaphoreType.DMA,
        ],
    )(logps2d)
    return out


# speedup vs baseline: 8.7430x; 8.7430x over previous
import functools

import jax
import jax.numpy as jnp
from jax import lax
from jax.experimental import pallas as pl
from jax.experimental.pallas import tpu as pltpu
from jax.experimental.pallas import tpu_sc as plsc

_NC = 2
_NS = 16
_NW = _NC * _NS
_BAND = 1024 // _NW


def kernel(hist, logps):
    S, B = hist.shape
    V = logps.shape[0]
    nslab = S + 1
    logps2d = logps.reshape(1, V)

    mesh = plsc.VectorSubcoreMesh(core_axis_name="c", subcore_axis_name="s")

    @functools.partial(
        pl.kernel,
        out_type=jax.ShapeDtypeStruct((nslab, B, V), jnp.float32),
        mesh=mesh,
        scratch_types=[
            pltpu.VMEM((_BAND, V), jnp.float32),
            pltpu.SemaphoreType.DMA,
            pltpu.SemaphoreType.DMA,
        ],
    )
    def _bcast(logps_hbm, out_hbm, buf, sem_fill, sem_out):
        c = lax.axis_index("c")
        s = lax.axis_index("s")
        wid = s * _NC + c
        row0 = wid * _BAND
        for r in range(_BAND):
            pltpu.make_async_copy(
                logps_hbm, buf.at[pl.ds(r, 1)], sem_fill
            ).start()
        for r in range(_BAND):
            pltpu.make_async_copy(
                logps_hbm, buf.at[pl.ds(r, 1)], sem_fill
            ).wait()
        for i in range(nslab):
            pltpu.make_async_copy(
                buf, out_hbm.at[i, pl.ds(row0, _BAND), :], sem_out
            ).start()
        for i in range(nslab):
            pltpu.make_async_copy(
                buf, out_hbm.at[i, pl.ds(row0, _BAND), :], sem_out
            ).wait()

    return _bcast(logps2d)


# speedup vs baseline: 14.6894x; 1.6801x over previous
import jax
import jax.numpy as jnp
from jax.experimental import pallas as pl


def _broadcast_kernel(logps_ref, out_ref):
    out_ref[...] = jnp.broadcast_to(logps_ref[...][:, None, :], out_ref.shape)


def kernel(hist, logps):
    S, B = hist.shape
    V = logps.shape[0]
    logps2d = logps.reshape(1, V)

    out = pl.pallas_call(
        _broadcast_kernel,
        grid=(S + 1,),
        in_specs=[pl.BlockSpec((1, V), lambda i: (0, 0))],
        out_specs=pl.BlockSpec((1, B, V), lambda i: (i, 0, 0)),
        out_shape=jax.ShapeDtypeStruct((S + 1, B, V), jnp.float32),
    )(logps2d)
    return out
